# initial kernel scaffold (unmeasured)
import jax
import jax.numpy as jnp
from jax import lax
from jax.experimental import pallas as pl
from jax.experimental.pallas import tpu as pltpu

N_DEV = 32
LOG2_N = 5


def kernel(x, w_mat):
    m, _k_loc = x.shape
    _, n = w_mat.shape
    m_per = m // N_DEV

    def body(x_ref, w_ref, y_ref, amax_ref,
             send_buf, recv_buf, send_sems, recv_sems, credit_sem,
             amax_recv, amax_ssems, amax_rsems):
        my = lax.axis_index("i")
        left = lax.rem(my - 1 + N_DEV, N_DEV)
        right = lax.rem(my + 1, N_DEV)

        barrier = pltpu.get_barrier_semaphore()
        for nbr in (left, right):
            pl.semaphore_signal(barrier, inc=1, device_id=(nbr,),
                                device_id_type=pl.DeviceIdType.MESH)
        pl.semaphore_wait(barrier, 2)

        def partial(c):
            xc = x_ref[pl.ds(c * m_per, m_per), :]
            return lax.dot_general(
                xc, w_ref[...], (((1,), (0,)), ((), ())),
                preferred_element_type=jnp.float32)

        acc = partial(left)
        send_descr = [None, None]
        for s in range(N_DEV - 1):
            slot = s % 2
            if s >= 2:
                pl.semaphore_wait(credit_sem, 1)
                send_descr[slot].wait_send()
            send_buf[slot] = acc.astype(jnp.bfloat16)
            rdma = pltpu.make_async_remote_copy(
                src_ref=send_buf.at[slot],
                dst_ref=recv_buf.at[slot],
                send_sem=send_sems.at[slot],
                recv_sem=recv_sems.at[slot],
                device_id=(right,),
                device_id_type=pl.DeviceIdType.MESH,
            )
            rdma.start()
            send_descr[slot] = rdma
            c_next = lax.rem(my - 2 - s + 2 * N_DEV, N_DEV)
            nxt = partial(c_next)
            rdma.wait_recv()
            acc = nxt + recv_buf[slot].astype(jnp.float32)
            if s <= N_DEV - 4:
                pl.semaphore_signal(credit_sem, inc=1, device_id=(left,),
                                    device_id_type=pl.DeviceIdType.MESH)
        send_descr[0].wait_send()
        send_descr[1].wait_send()

        y_ref[...] = acc

        amax_ref[0, 0] = jnp.max(jnp.abs(acc))
        for t in range(LOG2_N):
            partner = my ^ (1 << t)
            rdma = pltpu.make_async_remote_copy(
                src_ref=amax_ref,
                dst_ref=amax_recv.at[t],
                send_sem=amax_ssems.at[t],
                recv_sem=amax_rsems.at[t],
                device_id=(partner,),
                device_id_type=pl.DeviceIdType.MESH,
            )
            rdma.start()
            rdma.wait_recv()
            rdma.wait_send()
            amax_ref[0, 0] = jnp.maximum(amax_ref[0, 0], amax_recv[t, 0, 0])

    y, amax = pl.pallas_call(
        body,
        out_shape=[
            jax.ShapeDtypeStruct((m_per, n), jnp.float32),
            jax.ShapeDtypeStruct((1, 1), jnp.float32),
        ],
        in_specs=[
            pl.BlockSpec(memory_space=pltpu.VMEM),
            pl.BlockSpec(memory_space=pltpu.VMEM),
        ],
        out_specs=[
            pl.BlockSpec(memory_space=pltpu.VMEM),
            pl.BlockSpec(memory_space=pltpu.VMEM),
        ],
        scratch_shapes=[
            pltpu.VMEM((2, m_per, n), jnp.bfloat16),
            pltpu.VMEM((2, m_per, n), jnp.bfloat16),
            pltpu.SemaphoreType.DMA((2,)),
            pltpu.SemaphoreType.DMA((2,)),
            pltpu.SemaphoreType.REGULAR,
            pltpu.VMEM((LOG2_N, 1, 1), jnp.float32),
            pltpu.SemaphoreType.DMA((LOG2_N,)),
            pltpu.SemaphoreType.DMA((LOG2_N,)),
        ],
        compiler_params=pltpu.CompilerParams(collective_id=0),
    )(x.astype(jnp.bfloat16), w_mat.astype(jnp.bfloat16))

    scale = amax[0, 0] / 448.0
    q = jnp.clip(y / scale, -448.0, 448.0).astype(jnp.float8_e4m3fn)
    return q.astype(jnp.float32) * scale


# baseline (device time: 1509662 ns/iter reference)
import jax
import jax.numpy as jnp
from jax import lax
from jax.experimental import pallas as pl
from jax.experimental.pallas import tpu as pltpu

N_DEV = 32
LOG2_N = 5


def kernel(x, w_mat):
    m, _k_loc = x.shape
    _, n = w_mat.shape
    m_per = m // N_DEV

    def body(x_ref, w_ref, y_ref, amax_ref,
             send_buf, recv_buf, send_sems, recv_sems, credit_sem,
             amax_recv, amax_ssems, amax_rsems):
        my = lax.axis_index("i")
        left = lax.rem(my - 1 + N_DEV, N_DEV)
        right = lax.rem(my + 1, N_DEV)

        barrier = pltpu.get_barrier_semaphore()
        for nbr in (left, right):
            pl.semaphore_signal(barrier, inc=1, device_id=(nbr,),
                                device_id_type=pl.DeviceIdType.MESH)
        pl.semaphore_wait(barrier, 2)

        def partial(c):
            xc = x_ref[pl.ds(c * m_per, m_per), :]
            return lax.dot_general(
                xc, w_ref[...], (((1,), (0,)), ((), ())),
                preferred_element_type=jnp.float32)

        acc = partial(left)
        send_descr = [None, None]
        for s in range(N_DEV - 1):
            slot = s % 2
            if s >= 2:
                pl.semaphore_wait(credit_sem, 1)
                send_descr[slot].wait_send()
            send_buf[slot] = acc
            rdma = pltpu.make_async_remote_copy(
                src_ref=send_buf.at[slot],
                dst_ref=recv_buf.at[slot],
                send_sem=send_sems.at[slot],
                recv_sem=recv_sems.at[slot],
                device_id=(right,),
                device_id_type=pl.DeviceIdType.MESH,
            )
            rdma.start()
            send_descr[slot] = rdma
            c_next = lax.rem(my - 2 - s + 2 * N_DEV, N_DEV)
            nxt = partial(c_next)
            rdma.wait_recv()
            acc = nxt + recv_buf[slot]
            if s <= N_DEV - 4:
                pl.semaphore_signal(credit_sem, inc=1, device_id=(left,),
                                    device_id_type=pl.DeviceIdType.MESH)
        send_descr[0].wait_send()
        send_descr[1].wait_send()

        y_ref[...] = acc

        amax_ref[...] = jnp.max(jnp.abs(acc)).reshape(1, 1)
        for t in range(LOG2_N):
            partner = my ^ (1 << t)
            rdma = pltpu.make_async_remote_copy(
                src_ref=amax_ref,
                dst_ref=amax_recv.at[t],
                send_sem=amax_ssems.at[t],
                recv_sem=amax_rsems.at[t],
                device_id=(partner,),
                device_id_type=pl.DeviceIdType.MESH,
            )
            rdma.start()
            rdma.wait_recv()
            rdma.wait_send()
            amax_ref[...] = jnp.maximum(amax_ref[...], amax_recv[t])

    y, amax = pl.pallas_call(
        body,
        out_shape=[
            jax.ShapeDtypeStruct((m_per, n), jnp.float32),
            jax.ShapeDtypeStruct((1, 1), jnp.float32),
        ],
        in_specs=[
            pl.BlockSpec(memory_space=pltpu.VMEM),
            pl.BlockSpec(memory_space=pltpu.VMEM),
        ],
        out_specs=[
            pl.BlockSpec(memory_space=pltpu.VMEM),
            pl.BlockSpec(memory_space=pltpu.VMEM),
        ],
        scratch_shapes=[
            pltpu.VMEM((2, m_per, n), jnp.float32),
            pltpu.VMEM((2, m_per, n), jnp.float32),
            pltpu.SemaphoreType.DMA((2,)),
            pltpu.SemaphoreType.DMA((2,)),
            pltpu.SemaphoreType.REGULAR,
            pltpu.VMEM((LOG2_N, 1, 1), jnp.float32),
            pltpu.SemaphoreType.DMA((LOG2_N,)),
            pltpu.SemaphoreType.DMA((LOG2_N,)),
        ],
        compiler_params=pltpu.CompilerParams(collective_id=0),
    )(x.astype(jnp.bfloat16), w_mat.astype(jnp.bfloat16))

    scale = amax[0, 0] / 448.0
    v = jnp.clip(y / scale, -448.0, 448.0)
    u = lax.bitcast_convert_type(v, jnp.uint32)
    r = (u + jnp.uint32(0x7FFFF) + ((u >> 20) & jnp.uint32(1))) & jnp.uint32(
        0xFFF00000)
    norm = lax.bitcast_convert_type(r, jnp.float32)
    sub = jnp.round(v * 512.0) * (1.0 / 512.0)
    q = jnp.where(jnp.abs(v) < 2.0 ** -6, sub, norm)
    return q * scale


# device time: 1161364 ns/iter; 1.2999x vs baseline; 1.2999x over previous
import jax
import jax.numpy as jnp
from jax import lax
from jax.experimental import pallas as pl
from jax.experimental.pallas import tpu as pltpu

N_DEV = 32
HALF = N_DEV // 2
LOG2_N = 5
H_BF16 = 8


def kernel(x, w_mat):
    m, _k_loc = x.shape
    _, n = w_mat.shape
    m_per = m // N_DEV

    def body(x_ref, w_ref, y_ref, amax_ref,
             r_send_b, r_recv_b, l_send_b, l_recv_b,
             r_send_f, r_recv_f, l_send_f, l_recv_f,
             r_ssems, r_rsems, l_ssems, l_rsems,
             r_fssems, r_frsems, l_fssems, l_frsems,
             credit_r, credit_l,
             amax_recv, amax_ssems, amax_rsems):
        my = lax.axis_index("i")
        left = lax.rem(my - 1 + N_DEV, N_DEV)
        right = lax.rem(my + 1, N_DEV)

        barrier = pltpu.get_barrier_semaphore()
        for nbr in (left, right):
            pl.semaphore_signal(barrier, inc=1, device_id=(nbr,),
                                device_id_type=pl.DeviceIdType.MESH)
        pl.semaphore_wait(barrier, 2)

        def partial(c):
            xc = x_ref[pl.ds(c * m_per, m_per), :]
            return lax.dot_general(
                xc, w_ref[...], (((1,), (0,)), ((), ())),
                preferred_element_type=jnp.float32)

        def rdma(src, dst, ssems, lsems, slot, tgt):
            return pltpu.make_async_remote_copy(
                src_ref=src.at[slot], dst_ref=dst.at[slot],
                send_sem=ssems.at[slot], recv_sem=lsems.at[slot],
                device_id=(tgt,), device_id_type=pl.DeviceIdType.MESH)

        def credit(sem, tgt):
            pl.semaphore_signal(sem, inc=1, device_id=(tgt,),
                                device_id_type=pl.DeviceIdType.MESH)

        acc_r = partial(lax.rem(my + HALF, N_DEV))
        acc_l = partial(lax.rem(my - HALF + 1 + N_DEV, N_DEV))
        r_descr = [None, None, None]
        l_descr = [None, None, None]
        y_r = y_l = p_own = None
        for s in range(HALF):
            bf16_hop = s < H_BF16
            slot = s % 2 if bf16_hop else 0
            r_send = r_send_b if bf16_hop else r_send_f
            r_recv = r_recv_b if bf16_hop else r_recv_f
            l_send = l_send_b if bf16_hop else l_send_f
            l_recv = l_recv_b if bf16_hop else l_recv_f
            r_sems = (r_ssems, r_rsems) if bf16_hop else (r_fssems, r_frsems)
            l_sems = (l_ssems, l_rsems) if bf16_hop else (l_fssems, l_frsems)
            if s >= 2 and s != H_BF16:
                pl.semaphore_wait(credit_r, 1)
                r_descr[slot if bf16_hop else 2].wait_send()
            r_send[slot] = acc_r.astype(jnp.bfloat16) if bf16_hop else acc_r
            rr = rdma(r_send, r_recv, r_sems[0], r_sems[1], slot, right)
            rr.start()
            r_descr[slot if bf16_hop else 2] = rr
            if s <= HALF - 2:
                if s >= 2 and s != H_BF16:
                    pl.semaphore_wait(credit_l, 1)
                    l_descr[slot if bf16_hop else 2].wait_send()
                l_send[slot] = acc_l.astype(jnp.bfloat16) if bf16_hop else acc_l
                ll = rdma(l_send, l_recv, l_sems[0], l_sems[1], slot, left)
                ll.start()
                l_descr[slot if bf16_hop else 2] = ll
            if s <= HALF - 2:
                p_r = partial(lax.rem(my + HALF - (s + 1), N_DEV))
            if s <= HALF - 3:
                p_l = partial(lax.rem(my - HALF + 1 + (s + 1) + N_DEV, N_DEV))
            if s == HALF - 1:
                p_own = partial(my)
            rr.wait_recv()
            if s <= HALF - 2:
                acc_r = p_r + r_recv[slot].astype(jnp.float32)
                if s <= H_BF16 - 3 or s >= H_BF16:
                    credit(credit_r, left)
            else:
                y_r = r_recv[slot].astype(jnp.float32)
            if s <= HALF - 2:
                ll.wait_recv()
                if s <= HALF - 3:
                    acc_l = p_l + l_recv[slot].astype(jnp.float32)
                    if s <= H_BF16 - 3 or s >= H_BF16:
                        credit(credit_l, right)
                else:
                    y_l = l_recv[slot].astype(jnp.float32)
        for d in r_descr + l_descr:
            d.wait_send()

        acc = p_own + y_r + y_l
        y_ref[...] = acc

        amax_ref[...] = jnp.max(jnp.abs(acc)).reshape(1, 1)
        for t in range(LOG2_N):
            partner = my ^ (1 << t)
            bfly = pltpu.make_async_remote_copy(
                src_ref=amax_ref,
                dst_ref=amax_recv.at[t],
                send_sem=amax_ssems.at[t],
                recv_sem=amax_rsems.at[t],
                device_id=(partner,),
                device_id_type=pl.DeviceIdType.MESH,
            )
            bfly.start()
            bfly.wait_recv()
            bfly.wait_send()
            amax_ref[...] = jnp.maximum(amax_ref[...], amax_recv[t])

    y, amax = pl.pallas_call(
        body,
        out_shape=[
            jax.ShapeDtypeStruct((m_per, n), jnp.float32),
            jax.ShapeDtypeStruct((1, 1), jnp.float32),
        ],
        in_specs=[
            pl.BlockSpec(memory_space=pltpu.VMEM),
            pl.BlockSpec(memory_space=pltpu.VMEM),
        ],
        out_specs=[
            pl.BlockSpec(memory_space=pltpu.VMEM),
            pl.BlockSpec(memory_space=pltpu.VMEM),
        ],
        scratch_shapes=[
            pltpu.VMEM((2, m_per, n), jnp.bfloat16),
            pltpu.VMEM((2, m_per, n), jnp.bfloat16),
            pltpu.VMEM((2, m_per, n), jnp.bfloat16),
            pltpu.VMEM((2, m_per, n), jnp.bfloat16),
            pltpu.VMEM((1, m_per, n), jnp.float32),
            pltpu.VMEM((1, m_per, n), jnp.float32),
            pltpu.VMEM((1, m_per, n), jnp.float32),
            pltpu.VMEM((1, m_per, n), jnp.float32),
            pltpu.SemaphoreType.DMA((2,)),
            pltpu.SemaphoreType.DMA((2,)),
            pltpu.SemaphoreType.DMA((2,)),
            pltpu.SemaphoreType.DMA((2,)),
            pltpu.SemaphoreType.DMA((1,)),
            pltpu.SemaphoreType.DMA((1,)),
            pltpu.SemaphoreType.DMA((1,)),
            pltpu.SemaphoreType.DMA((1,)),
            pltpu.SemaphoreType.REGULAR,
            pltpu.SemaphoreType.REGULAR,
            pltpu.VMEM((LOG2_N, 1, 1), jnp.float32),
            pltpu.SemaphoreType.DMA((LOG2_N,)),
            pltpu.SemaphoreType.DMA((LOG2_N,)),
        ],
        compiler_params=pltpu.CompilerParams(
            collective_id=0, vmem_limit_bytes=100 * 1024 * 1024),
    )(x.astype(jnp.bfloat16), w_mat.astype(jnp.bfloat16))

    scale = amax[0, 0] / 448.0
    v = jnp.clip(y / scale, -448.0, 448.0)
    u = lax.bitcast_convert_type(v, jnp.uint32)
    r = (u + jnp.uint32(0x7FFFF) + ((u >> 20) & jnp.uint32(1))) & jnp.uint32(
        0xFFF00000)
    norm = lax.bitcast_convert_type(r, jnp.float32)
    sub = jnp.round(v * 512.0) * (1.0 / 512.0)
    q = jnp.where(jnp.abs(v) < 2.0 ** -6, sub, norm)
    return q * scale


# device time: 626685 ns/iter; 2.4090x vs baseline; 1.8532x over previous
import jax
import jax.numpy as jnp
from jax import lax
from jax.experimental import pallas as pl
from jax.experimental.pallas import tpu as pltpu

N_DEV = 32
HALF = N_DEV // 2
LOG2_N = 5

RING_PERM = [0, 8, 16, 24, 27, 28, 31, 23, 20, 19, 11, 12, 15, 7, 4, 3,
             2, 5, 6, 14, 13, 10, 18, 21, 22, 30, 29, 26, 25, 17, 9, 1]
RING_INV = [0] * N_DEV
for _k, _l in enumerate(RING_PERM):
    RING_INV[_l] = _k
H_BF16 = 8


def kernel(x, w_mat):
    m, _k_loc = x.shape
    _, n = w_mat.shape
    m_per = m // N_DEV

    def body(x_ref, w_ref, perm_ref, inv_ref, y_ref, amax_ref,
             r_send_b, r_recv_b, l_send_b, l_recv_b,
             r_send_f, r_recv_f, l_send_f, l_recv_f,
             r_ssems, r_rsems, l_ssems, l_rsems,
             r_fssems, r_frsems, l_fssems, l_frsems,
             credit_r, credit_l,
             amax_recv, amax_ssems, amax_rsems):
        my = lax.axis_index("i")
        k = inv_ref[my]
        left = perm_ref[lax.rem(k - 1 + N_DEV, N_DEV)]
        right = perm_ref[lax.rem(k + 1, N_DEV)]

        barrier = pltpu.get_barrier_semaphore()
        for nbr in (left, right):
            pl.semaphore_signal(barrier, inc=1, device_id=(nbr,),
                                device_id_type=pl.DeviceIdType.MESH)
        pl.semaphore_wait(barrier, 2)

        def partial(c):
            xc = x_ref[pl.ds(c * m_per, m_per), :]
            return lax.dot_general(
                xc, w_ref[...], (((1,), (0,)), ((), ())),
                preferred_element_type=jnp.float32)

        def rdma(src, dst, ssems, lsems, slot, tgt):
            return pltpu.make_async_remote_copy(
                src_ref=src.at[slot], dst_ref=dst.at[slot],
                send_sem=ssems.at[slot], recv_sem=lsems.at[slot],
                device_id=(tgt,), device_id_type=pl.DeviceIdType.MESH)

        def credit(sem, tgt):
            pl.semaphore_signal(sem, inc=1, device_id=(tgt,),
                                device_id_type=pl.DeviceIdType.MESH)

        def chunk_r(s):
            return perm_ref[lax.rem(k + HALF - s + N_DEV, N_DEV)]

        def chunk_l(s):
            return perm_ref[lax.rem(k - HALF + 1 + s + N_DEV, N_DEV)]

        acc_r = partial(chunk_r(0))
        acc_l = partial(chunk_l(0))
        r_descr = [None, None, None]
        l_descr = [None, None, None]
        y_r = y_l = p_own = None
        for s in range(HALF):
            bf16_hop = s < H_BF16
            slot = s % 2 if bf16_hop else 0
            r_send = r_send_b if bf16_hop else r_send_f
            r_recv = r_recv_b if bf16_hop else r_recv_f
            l_send = l_send_b if bf16_hop else l_send_f
            l_recv = l_recv_b if bf16_hop else l_recv_f
            r_sems = (r_ssems, r_rsems) if bf16_hop else (r_fssems, r_frsems)
            l_sems = (l_ssems, l_rsems) if bf16_hop else (l_fssems, l_frsems)
            if s >= 2 and s != H_BF16:
                pl.semaphore_wait(credit_r, 1)
                r_descr[slot if bf16_hop else 2].wait_send()
            r_send[slot] = acc_r.astype(jnp.bfloat16) if bf16_hop else acc_r
            rr = rdma(r_send, r_recv, r_sems[0], r_sems[1], slot, right)
            rr.start()
            r_descr[slot if bf16_hop else 2] = rr
            if s <= HALF - 2:
                if s >= 2 and s != H_BF16:
                    pl.semaphore_wait(credit_l, 1)
                    l_descr[slot if bf16_hop else 2].wait_send()
                l_send[slot] = acc_l.astype(jnp.bfloat16) if bf16_hop else acc_l
                ll = rdma(l_send, l_recv, l_sems[0], l_sems[1], slot, left)
                ll.start()
                l_descr[slot if bf16_hop else 2] = ll
            if s <= HALF - 2:
                p_r = partial(chunk_r(s + 1))
            if s <= HALF - 3:
                p_l = partial(chunk_l(s + 1))
            if s == HALF - 1:
                p_own = partial(my)
            rr.wait_recv()
            if s <= HALF - 2:
                acc_r = p_r + r_recv[slot].astype(jnp.float32)
                if s <= H_BF16 - 3 or s >= H_BF16:
                    credit(credit_r, left)
            else:
                y_r = r_recv[slot].astype(jnp.float32)
            if s <= HALF - 2:
                ll.wait_recv()
                if s <= HALF - 3:
                    acc_l = p_l + l_recv[slot].astype(jnp.float32)
                    if s <= H_BF16 - 3 or s >= H_BF16:
                        credit(credit_l, right)
                else:
                    y_l = l_recv[slot].astype(jnp.float32)
        for d in r_descr + l_descr:
            d.wait_send()

        acc = p_own + y_r + y_l
        y_ref[...] = acc

        amax_ref[...] = jnp.max(jnp.abs(acc)).reshape(1, 1)
        for t in range(LOG2_N):
            partner = my ^ (1 << t)
            bfly = pltpu.make_async_remote_copy(
                src_ref=amax_ref,
                dst_ref=amax_recv.at[t],
                send_sem=amax_ssems.at[t],
                recv_sem=amax_rsems.at[t],
                device_id=(partner,),
                device_id_type=pl.DeviceIdType.MESH,
            )
            bfly.start()
            bfly.wait_recv()
            bfly.wait_send()
            amax_ref[...] = jnp.maximum(amax_ref[...], amax_recv[t])

    y, amax = pl.pallas_call(
        body,
        out_shape=[
            jax.ShapeDtypeStruct((m_per, n), jnp.float32),
            jax.ShapeDtypeStruct((1, 1), jnp.float32),
        ],
        in_specs=[
            pl.BlockSpec(memory_space=pltpu.VMEM),
            pl.BlockSpec(memory_space=pltpu.VMEM),
            pl.BlockSpec(memory_space=pltpu.SMEM),
            pl.BlockSpec(memory_space=pltpu.SMEM),
        ],
        out_specs=[
            pl.BlockSpec(memory_space=pltpu.VMEM),
            pl.BlockSpec(memory_space=pltpu.VMEM),
        ],
        scratch_shapes=[
            pltpu.VMEM((2, m_per, n), jnp.bfloat16),
            pltpu.VMEM((2, m_per, n), jnp.bfloat16),
            pltpu.VMEM((2, m_per, n), jnp.bfloat16),
            pltpu.VMEM((2, m_per, n), jnp.bfloat16),
            pltpu.VMEM((1, m_per, n), jnp.float32),
            pltpu.VMEM((1, m_per, n), jnp.float32),
            pltpu.VMEM((1, m_per, n), jnp.float32),
            pltpu.VMEM((1, m_per, n), jnp.float32),
            pltpu.SemaphoreType.DMA((2,)),
            pltpu.SemaphoreType.DMA((2,)),
            pltpu.SemaphoreType.DMA((2,)),
            pltpu.SemaphoreType.DMA((2,)),
            pltpu.SemaphoreType.DMA((1,)),
            pltpu.SemaphoreType.DMA((1,)),
            pltpu.SemaphoreType.DMA((1,)),
            pltpu.SemaphoreType.DMA((1,)),
            pltpu.SemaphoreType.REGULAR,
            pltpu.SemaphoreType.REGULAR,
            pltpu.VMEM((LOG2_N, 1, 1), jnp.float32),
            pltpu.SemaphoreType.DMA((LOG2_N,)),
            pltpu.SemaphoreType.DMA((LOG2_N,)),
        ],
        compiler_params=pltpu.CompilerParams(
            collective_id=0, vmem_limit_bytes=100 * 1024 * 1024),
    )(x.astype(jnp.bfloat16), w_mat.astype(jnp.bfloat16),
      jnp.array(RING_PERM, jnp.int32), jnp.array(RING_INV, jnp.int32))

    scale = amax[0, 0] / 448.0
    v = jnp.clip(y / scale, -448.0, 448.0)
    u = lax.bitcast_convert_type(v, jnp.uint32)
    r = (u + jnp.uint32(0x7FFFF) + ((u >> 20) & jnp.uint32(1))) & jnp.uint32(
        0xFFF00000)
    norm = lax.bitcast_convert_type(r, jnp.float32)
    sub = jnp.round(v * 512.0) * (1.0 / 512.0)
    q = jnp.where(jnp.abs(v) < 2.0 ** -6, sub, norm)
    return q * scale


# device time: 445964 ns/iter; 3.3852x vs baseline; 1.4052x over previous
import jax
import jax.numpy as jnp
from jax import lax
from jax.experimental import pallas as pl
from jax.experimental.pallas import tpu as pltpu

N_DEV = 32
HALF = N_DEV // 2
LOG2_N = 5

RING_PERM = [0, 8, 16, 24, 27, 28, 31, 23, 20, 19, 11, 12, 15, 7, 4, 3,
             2, 5, 6, 14, 13, 10, 18, 21, 22, 30, 29, 26, 25, 17, 9, 1]
RING_INV = [0] * N_DEV
for _k, _l in enumerate(RING_PERM):
    RING_INV[_l] = _k
H_BF16 = 16


def kernel(x, w_mat):
    m, _k_loc = x.shape
    _, n = w_mat.shape
    m_per = m // N_DEV

    def body(x_ref, w_ref, perm_ref, inv_ref, y_ref, amax_ref,
             r_send_b, r_recv_b, l_send_b, l_recv_b,
             r_send_f, r_recv_f, l_send_f, l_recv_f,
             r_ssems, r_rsems, l_ssems, l_rsems,
             r_fssems, r_frsems, l_fssems, l_frsems,
             credit_r, credit_l,
             amax_recv, amax_ssems, amax_rsems):
        my = lax.axis_index("i")
        k = inv_ref[my]
        left = perm_ref[lax.rem(k - 1 + N_DEV, N_DEV)]
        right = perm_ref[lax.rem(k + 1, N_DEV)]

        barrier = pltpu.get_barrier_semaphore()
        for nbr in (left, right):
            pl.semaphore_signal(barrier, inc=1, device_id=(nbr,),
                                device_id_type=pl.DeviceIdType.MESH)
        pl.semaphore_wait(barrier, 2)

        def partial(c):
            xc = x_ref[pl.ds(c * m_per, m_per), :]
            return lax.dot_general(
                xc, w_ref[...], (((1,), (0,)), ((), ())),
                preferred_element_type=jnp.float32)

        def rdma(src, dst, ssems, lsems, slot, tgt):
            return pltpu.make_async_remote_copy(
                src_ref=src.at[slot], dst_ref=dst.at[slot],
                send_sem=ssems.at[slot], recv_sem=lsems.at[slot],
                device_id=(tgt,), device_id_type=pl.DeviceIdType.MESH)

        def credit(sem, tgt):
            pl.semaphore_signal(sem, inc=1, device_id=(tgt,),
                                device_id_type=pl.DeviceIdType.MESH)

        def needs_credit(s, last_send):
            if s < H_BF16:
                return s + 2 < H_BF16 and s + 2 <= last_send
            return s + 1 <= last_send

        def chunk_r(s):
            return perm_ref[lax.rem(k + HALF - s + N_DEV, N_DEV)]

        def chunk_l(s):
            return perm_ref[lax.rem(k - HALF + 1 + s + N_DEV, N_DEV)]

        acc_r = partial(chunk_r(0))
        acc_l = partial(chunk_l(0))
        r_descr = [None, None, None]
        l_descr = [None, None, None]
        y_r = y_l = p_own = None
        for s in range(HALF):
            bf16_hop = s < H_BF16
            slot = s % 2 if bf16_hop else 0
            r_send = r_send_b if bf16_hop else r_send_f
            r_recv = r_recv_b if bf16_hop else r_recv_f
            l_send = l_send_b if bf16_hop else l_send_f
            l_recv = l_recv_b if bf16_hop else l_recv_f
            r_sems = (r_ssems, r_rsems) if bf16_hop else (r_fssems, r_frsems)
            l_sems = (l_ssems, l_rsems) if bf16_hop else (l_fssems, l_frsems)
            if s >= 2 and s != H_BF16:
                pl.semaphore_wait(credit_r, 1)
                r_descr[slot if bf16_hop else 2].wait_send()
            r_send[slot] = acc_r.astype(jnp.bfloat16) if bf16_hop else acc_r
            rr = rdma(r_send, r_recv, r_sems[0], r_sems[1], slot, right)
            rr.start()
            r_descr[slot if bf16_hop else 2] = rr
            if s <= HALF - 2:
                if s >= 2 and s != H_BF16:
                    pl.semaphore_wait(credit_l, 1)
                    l_descr[slot if bf16_hop else 2].wait_send()
                l_send[slot] = acc_l.astype(jnp.bfloat16) if bf16_hop else acc_l
                ll = rdma(l_send, l_recv, l_sems[0], l_sems[1], slot, left)
                ll.start()
                l_descr[slot if bf16_hop else 2] = ll
            if s <= HALF - 2:
                p_r = partial(chunk_r(s + 1))
            if s <= HALF - 3:
                p_l = partial(chunk_l(s + 1))
            if s == HALF - 1:
                p_own = partial(my)
            rr.wait_recv()
            if s <= HALF - 2:
                acc_r = p_r + r_recv[slot].astype(jnp.float32)
                if needs_credit(s, HALF - 1):
                    credit(credit_r, left)
            else:
                y_r = r_recv[slot].astype(jnp.float32)
            if s <= HALF - 2:
                ll.wait_recv()
                if s <= HALF - 3:
                    acc_l = p_l + l_recv[slot].astype(jnp.float32)
                    if needs_credit(s, HALF - 2):
                        credit(credit_l, right)
                else:
                    y_l = l_recv[slot].astype(jnp.float32)
        for d in r_descr + l_descr:
            if d is not None:
                d.wait_send()

        acc = p_own + y_r + y_l
        y_ref[...] = acc

        amax_ref[...] = jnp.max(jnp.abs(acc)).reshape(1, 1)
        for t in range(LOG2_N):
            partner = my ^ (1 << t)
            bfly = pltpu.make_async_remote_copy(
                src_ref=amax_ref,
                dst_ref=amax_recv.at[t],
                send_sem=amax_ssems.at[t],
                recv_sem=amax_rsems.at[t],
                device_id=(partner,),
                device_id_type=pl.DeviceIdType.MESH,
            )
            bfly.start()
            bfly.wait_recv()
            bfly.wait_send()
            amax_ref[...] = jnp.maximum(amax_ref[...], amax_recv[t])

    y, amax = pl.pallas_call(
        body,
        out_shape=[
            jax.ShapeDtypeStruct((m_per, n), jnp.float32),
            jax.ShapeDtypeStruct((1, 1), jnp.float32),
        ],
        in_specs=[
            pl.BlockSpec(memory_space=pltpu.VMEM),
            pl.BlockSpec(memory_space=pltpu.VMEM),
            pl.BlockSpec(memory_space=pltpu.SMEM),
            pl.BlockSpec(memory_space=pltpu.SMEM),
        ],
        out_specs=[
            pl.BlockSpec(memory_space=pltpu.VMEM),
            pl.BlockSpec(memory_space=pltpu.VMEM),
        ],
        scratch_shapes=[
            pltpu.VMEM((2, m_per, n), jnp.bfloat16),
            pltpu.VMEM((2, m_per, n), jnp.bfloat16),
            pltpu.VMEM((2, m_per, n), jnp.bfloat16),
            pltpu.VMEM((2, m_per, n), jnp.bfloat16),
            pltpu.VMEM((1, m_per, n), jnp.float32),
            pltpu.VMEM((1, m_per, n), jnp.float32),
            pltpu.VMEM((1, m_per, n), jnp.float32),
            pltpu.VMEM((1, m_per, n), jnp.float32),
            pltpu.SemaphoreType.DMA((2,)),
            pltpu.SemaphoreType.DMA((2,)),
            pltpu.SemaphoreType.DMA((2,)),
            pltpu.SemaphoreType.DMA((2,)),
            pltpu.SemaphoreType.DMA((1,)),
            pltpu.SemaphoreType.DMA((1,)),
            pltpu.SemaphoreType.DMA((1,)),
            pltpu.SemaphoreType.DMA((1,)),
            pltpu.SemaphoreType.REGULAR,
            pltpu.SemaphoreType.REGULAR,
            pltpu.VMEM((LOG2_N, 1, 1), jnp.float32),
            pltpu.SemaphoreType.DMA((LOG2_N,)),
            pltpu.SemaphoreType.DMA((LOG2_N,)),
        ],
        compiler_params=pltpu.CompilerParams(
            collective_id=0, vmem_limit_bytes=100 * 1024 * 1024),
    )(x.astype(jnp.bfloat16), w_mat.astype(jnp.bfloat16),
      jnp.array(RING_PERM, jnp.int32), jnp.array(RING_INV, jnp.int32))

    scale = amax[0, 0] / 448.0
    v = jnp.clip(y / scale, -448.0, 448.0)
    u = lax.bitcast_convert_type(v, jnp.uint32)
    r = (u + jnp.uint32(0x7FFFF) + ((u >> 20) & jnp.uint32(1))) & jnp.uint32(
        0xFFF00000)
    norm = lax.bitcast_convert_type(r, jnp.float32)
    sub = jnp.round(v * 512.0) * (1.0 / 512.0)
    q = jnp.where(jnp.abs(v) < 2.0 ** -6, sub, norm)
    return q * scale


# device time: 440763 ns/iter; 3.4251x vs baseline; 1.0118x over previous
import jax
import jax.numpy as jnp
from jax import lax
from jax.experimental import pallas as pl
from jax.experimental.pallas import tpu as pltpu

N_DEV = 32
HALF = N_DEV // 2

RING_PERM = [0, 8, 16, 24, 27, 28, 31, 23, 20, 19, 11, 12, 15, 7, 4, 3,
             2, 5, 6, 14, 13, 10, 18, 21, 22, 30, 29, 26, 25, 17, 9, 1]
RING_INV = [0] * N_DEV
for _k, _l in enumerate(RING_PERM):
    RING_INV[_l] = _k


def kernel(x, w_mat):
    m, _k_loc = x.shape
    _, n = w_mat.shape
    m_per = m // N_DEV

    def body(x_ref, w_ref, perm_ref, inv_ref, y_ref, amax_ref,
             r_send_b, r_recv_b, l_send_b, l_recv_b,
             r_ssems, r_rsems, l_ssems, l_rsems,
             credit_r, credit_l,
             amax_recv, amax_ssems, amax_rsems):
        my = lax.axis_index("i")
        k = inv_ref[my]
        left = perm_ref[lax.rem(k - 1 + N_DEV, N_DEV)]
        right = perm_ref[lax.rem(k + 1, N_DEV)]

        barrier = pltpu.get_barrier_semaphore()
        for nbr in (left, right):
            pl.semaphore_signal(barrier, inc=1, device_id=(nbr,),
                                device_id_type=pl.DeviceIdType.MESH)
        pl.semaphore_wait(barrier, 2)

        def partial(c):
            xc = x_ref[pl.ds(c * m_per, m_per), :]
            return lax.dot_general(
                xc, w_ref[...], (((1,), (0,)), ((), ())),
                preferred_element_type=jnp.float32)

        def rdma(src, dst, ssems, lsems, slot, tgt):
            return pltpu.make_async_remote_copy(
                src_ref=src.at[slot], dst_ref=dst.at[slot],
                send_sem=ssems.at[slot], recv_sem=lsems.at[slot],
                device_id=(tgt,), device_id_type=pl.DeviceIdType.MESH)

        def credit(sem, tgt):
            pl.semaphore_signal(sem, inc=1, device_id=(tgt,),
                                device_id_type=pl.DeviceIdType.MESH)

        def needs_credit(s, last_send):
            return s + 2 <= last_send

        def chunk_r(s):
            return perm_ref[lax.rem(k + HALF - s + N_DEV, N_DEV)]

        def chunk_l(s):
            return perm_ref[lax.rem(k - HALF + 1 + s + N_DEV, N_DEV)]

        acc_r = partial(chunk_r(0))
        acc_l = partial(chunk_l(0))
        r_descr = [None, None]
        l_descr = [None, None]
        y_r = y_l = p_own = None
        for s in range(HALF):
            slot = s % 2
            if s >= 2:
                pl.semaphore_wait(credit_r, 1)
                r_descr[slot].wait_send()
            r_send_b[slot] = acc_r.astype(jnp.bfloat16)
            rr = rdma(r_send_b, r_recv_b, r_ssems, r_rsems, slot, right)
            rr.start()
            r_descr[slot] = rr
            if s <= HALF - 2:
                if s >= 2:
                    pl.semaphore_wait(credit_l, 1)
                    l_descr[slot].wait_send()
                l_send_b[slot] = acc_l.astype(jnp.bfloat16)
                ll = rdma(l_send_b, l_recv_b, l_ssems, l_rsems, slot, left)
                ll.start()
                l_descr[slot] = ll
            if s <= HALF - 2:
                p_r = partial(chunk_r(s + 1))
            if s <= HALF - 3:
                p_l = partial(chunk_l(s + 1))
            if s == HALF - 1:
                p_own = partial(my)
            rr.wait_recv()
            if s <= HALF - 2:
                acc_r = p_r + r_recv_b[slot].astype(jnp.float32)
                if needs_credit(s, HALF - 1):
                    credit(credit_r, left)
            else:
                y_r = r_recv_b[slot].astype(jnp.float32)
            if s <= HALF - 2:
                ll.wait_recv()
                if s <= HALF - 3:
                    acc_l = p_l + l_recv_b[slot].astype(jnp.float32)
                    if needs_credit(s, HALF - 2):
                        credit(credit_l, right)
                else:
                    y_l = l_recv_b[slot].astype(jnp.float32)
        for d in r_descr + l_descr:
            d.wait_send()

        acc = p_own + y_r + y_l
        y_ref[...] = acc

        amax_ref[...] = jnp.max(jnp.abs(acc)).reshape(1, 1)
        own = amax_ref[...]
        descrs = []
        for j in range(N_DEV):
            d = pltpu.make_async_remote_copy(
                src_ref=amax_ref,
                dst_ref=amax_recv.at[my],
                send_sem=amax_ssems.at[j],
                recv_sem=amax_rsems.at[my],
                device_id=(j,),
                device_id_type=pl.DeviceIdType.MESH,
            )
            descrs.append(d)
            pl.when(my != j)(d.start)
        m = own
        for j in range(N_DEV):
            w = pltpu.make_async_remote_copy(
                src_ref=amax_ref,
                dst_ref=amax_recv.at[j],
                send_sem=amax_ssems.at[j],
                recv_sem=amax_rsems.at[j],
                device_id=(j,),
                device_id_type=pl.DeviceIdType.MESH,
            )
            pl.when(my != j)(w.wait_recv)
            m = jnp.maximum(m, jnp.where(my == j, own, amax_recv[j]))
        for j in range(N_DEV):
            pl.when(my != j)(descrs[j].wait_send)
        amax_ref[...] = m

    y, amax = pl.pallas_call(
        body,
        out_shape=[
            jax.ShapeDtypeStruct((m_per, n), jnp.float32),
            jax.ShapeDtypeStruct((1, 1), jnp.float32),
        ],
        in_specs=[
            pl.BlockSpec(memory_space=pltpu.VMEM),
            pl.BlockSpec(memory_space=pltpu.VMEM),
            pl.BlockSpec(memory_space=pltpu.SMEM),
            pl.BlockSpec(memory_space=pltpu.SMEM),
        ],
        out_specs=[
            pl.BlockSpec(memory_space=pltpu.VMEM),
            pl.BlockSpec(memory_space=pltpu.VMEM),
        ],
        scratch_shapes=[
            pltpu.VMEM((2, m_per, n), jnp.bfloat16),
            pltpu.VMEM((2, m_per, n), jnp.bfloat16),
            pltpu.VMEM((2, m_per, n), jnp.bfloat16),
            pltpu.VMEM((2, m_per, n), jnp.bfloat16),
            pltpu.SemaphoreType.DMA((2,)),
            pltpu.SemaphoreType.DMA((2,)),
            pltpu.SemaphoreType.DMA((2,)),
            pltpu.SemaphoreType.DMA((2,)),
            pltpu.SemaphoreType.REGULAR,
            pltpu.SemaphoreType.REGULAR,
            pltpu.VMEM((N_DEV, 1, 1), jnp.float32),
            pltpu.SemaphoreType.DMA((N_DEV,)),
            pltpu.SemaphoreType.DMA((N_DEV,)),
        ],
        compiler_params=pltpu.CompilerParams(
            collective_id=0, vmem_limit_bytes=100 * 1024 * 1024),
    )(x.astype(jnp.bfloat16), w_mat.astype(jnp.bfloat16),
      jnp.array(RING_PERM, jnp.int32), jnp.array(RING_INV, jnp.int32))

    scale = amax[0, 0] / 448.0
    v = jnp.clip(y / scale, -448.0, 448.0)
    u = lax.bitcast_convert_type(v, jnp.uint32)
    r = (u + jnp.uint32(0x7FFFF) + ((u >> 20) & jnp.uint32(1))) & jnp.uint32(
        0xFFF00000)
    norm = lax.bitcast_convert_type(r, jnp.float32)
    sub = jnp.round(v * 512.0) * (1.0 / 512.0)
    q = jnp.where(jnp.abs(v) < 2.0 ** -6, sub, norm)
    return q * scale
